# baseline (speedup 1.0000x reference)
import functools

import jax
import jax.numpy as jnp
from jax import lax
from jax.experimental import pallas as pl
from jax.experimental.pallas import tpu as pltpu
from jax.experimental.pallas import tpu_sc as plsc

_N = 4096
_W = 4096
_NWORK = 32
_PER_W = _N // _NWORK
_C = 4
_NCHUNK = _PER_W // _C
_L = 16
_NL = 5


def _chain_body(x0_hbm, idx0_hbm, idx1_hbm, mmb_hbm, out_hbm, xa_hbm, xb_hbm,
                idx0_v, idx1_v, mmb_v, a_v, b_v, o_v,
                sem_a0, sem_a1, sem_b0, sem_b1, sem_o0, sem_o1, bar_sem):
    cid = lax.axis_index("c")
    wid = lax.axis_index("s") * 2 + cid
    sem_a = (sem_a0, sem_a1)
    sem_b = (sem_b0, sem_b1)
    sem_o = (sem_o0, sem_o1)
    srcs = [x0_hbm, xa_hbm, xb_hbm, xa_hbm, xb_hbm]
    dsts = [xa_hbm, xb_hbm, xa_hbm, xb_hbm, out_hbm]

    for layer in range(_NL):
        src = srcs[layer]
        dst = dsts[layer]
        pltpu.sync_copy(idx0_hbm.at[layer * _NWORK + wid], idx0_v)
        pltpu.sync_copy(idx1_hbm.at[layer * _NWORK + wid], idx1_v)
        pltpu.sync_copy(mmb_hbm.at[layer * _NWORK + wid], mmb_v)

        def gather(g, p, src=src):
            pltpu.async_copy(src.at[idx0_v.at[g]], a_v.at[p], sem_a[p])
            pltpu.async_copy(src.at[idx1_v.at[g]], b_v.at[p], sem_b[p])

        def compute(g, p, src=src, dst=dst):
            pltpu.make_async_copy(src.at[pl.ds(0, _C)], a_v.at[p], sem_a[p]).wait()
            pltpu.make_async_copy(src.at[pl.ds(0, _C)], b_v.at[p], sem_b[p]).wait()
            for c in range(_C):
                mm = mmb_v[g * _C + c, :]
                nm = jnp.bitwise_not(mm)

                def wbody(w, _, c=c, mm=mm, nm=nm):
                    sl = pl.ds(w * _L, _L)
                    a = a_v[p, c, sl]
                    b = b_v[p, c, sl]
                    o_v[p, c, sl] = ((a ^ mm) & (b ^ mm)) ^ nm
                    return 0

                lax.fori_loop(0, _W // _L, wbody, 0, unroll=8)
            base = wid * _PER_W + g * _C
            pltpu.async_copy(o_v.at[p], dst.at[pl.ds(base, _C)], sem_o[p])

        gather(0, 0)

        def chunk_body(h, carry, gather=gather, compute=compute, dst=dst):
            g0 = h * 2
            for q in range(2):
                g = g0 + q
                p = q
                nxt = g + 1
                if q == 0:
                    gather(nxt, 1)
                else:
                    @pl.when(nxt < _NCHUNK)
                    def _():
                        gather(nxt, 0)

                @pl.when(g >= 2)
                def _():
                    pltpu.make_async_copy(
                        o_v.at[p], dst.at[pl.ds(0, _C)], sem_o[p]).wait()

                compute(g, p)
            return carry

        lax.fori_loop(0, _NCHUNK // 2, chunk_body, 0)
        pltpu.make_async_copy(o_v.at[0], dst.at[pl.ds(0, _C)], sem_o[0]).wait()
        pltpu.make_async_copy(o_v.at[1], dst.at[pl.ds(0, _C)], sem_o[1]).wait()

        if layer < _NL - 1:
            plsc.subcore_barrier()
            pl.semaphore_signal(bar_sem, 1, core_index=1 - cid)
            pl.semaphore_wait(bar_sem, 1)


_sc_chain = functools.partial(
    pl.kernel,
    mesh=plsc.VectorSubcoreMesh(core_axis_name="c", subcore_axis_name="s"),
    out_type=(
        jax.ShapeDtypeStruct((_N, _W), jnp.int32),
        jax.ShapeDtypeStruct((_N, _W), jnp.int32),
        jax.ShapeDtypeStruct((_N, _W), jnp.int32),
    ),
    scratch_types=[
        pltpu.VMEM((_NCHUNK, _C), jnp.int32),
        pltpu.VMEM((_NCHUNK, _C), jnp.int32),
        pltpu.VMEM((_PER_W, _L), jnp.int32),
        pltpu.VMEM((2, _C, _W), jnp.int32),
        pltpu.VMEM((2, _C, _W), jnp.int32),
        pltpu.VMEM((2, _C, _W), jnp.int32),
        pltpu.SemaphoreType.DMA,
        pltpu.SemaphoreType.DMA,
        pltpu.SemaphoreType.DMA,
        pltpu.SemaphoreType.DMA,
        pltpu.SemaphoreType.DMA,
        pltpu.SemaphoreType.DMA,
        pltpu.SemaphoreType.REGULAR,
    ],
)(_chain_body)


def kernel(input_bitarrays, output_shape, adj_logits_0, nor_logits_0, adj_temp_0, nor_temp_0, adj_logits_1, nor_logits_1, adj_temp_1, nor_temp_1, adj_logits_2, nor_logits_2, adj_temp_2, nor_temp_2, adj_logits_3, nor_logits_3, adj_temp_3, nor_temp_3, adj_logits_4, nor_logits_4, adj_temp_4, nor_temp_4):
    params = {
        'adj_logits_0': adj_logits_0, 'nor_logits_0': nor_logits_0,
        'adj_temp_0': adj_temp_0, 'nor_temp_0': nor_temp_0,
        'adj_logits_1': adj_logits_1, 'nor_logits_1': nor_logits_1,
        'adj_temp_1': adj_temp_1, 'nor_temp_1': nor_temp_1,
        'adj_logits_2': adj_logits_2, 'nor_logits_2': nor_logits_2,
        'adj_temp_2': adj_temp_2, 'nor_temp_2': nor_temp_2,
        'adj_logits_3': adj_logits_3, 'nor_logits_3': nor_logits_3,
        'adj_temp_3': adj_temp_3, 'nor_temp_3': nor_temp_3,
        'adj_logits_4': adj_logits_4, 'nor_logits_4': nor_logits_4,
        'adj_temp_4': adj_temp_4, 'nor_temp_4': nor_temp_4,
    }
    key = jax.random.key(42)
    idx0s, idx1s, mms = [], [], []
    for i in range(_NL):
        al = params[f'adj_logits_{i}']
        at = params[f'adj_temp_{i}']
        nl = params[f'nor_logits_{i}']
        nt = params[f'nor_temp_{i}']
        k = jax.random.fold_in(key, i)
        k1, k2, k3 = jax.random.split(k, 3)
        logits = al * at
        idx0s.append(jax.random.categorical(k1, logits, axis=-1).astype(jnp.int32))
        idx1s.append(jax.random.categorical(k2, logits, axis=-1).astype(jnp.int32))
        nor_mask = jax.random.bernoulli(k3, jax.nn.sigmoid(nl * nt))
        mms.append(jnp.where(nor_mask, jnp.int32(-1), jnp.int32(0)))
    idx0c = jnp.stack(idx0s).reshape(_NL * _NWORK, _NCHUNK, _C)
    idx1c = jnp.stack(idx1s).reshape(_NL * _NWORK, _NCHUNK, _C)
    mmb = jnp.broadcast_to(jnp.stack(mms)[:, :, None], (_NL, _N, _L))
    mmb = mmb.reshape(_NL * _NWORK, _PER_W, _L)
    x0 = jnp.bitwise_or(input_bitarrays, jnp.int32(0) * output_shape)
    out, _, _ = _sc_chain(x0, idx0c, idx1c, mmb)
    return out


# speedup vs baseline: 1.1550x; 1.1550x over previous
import functools

import jax
import jax.numpy as jnp
from jax import lax
from jax.experimental import pallas as pl
from jax.experimental.pallas import tpu as pltpu
from jax.experimental.pallas import tpu_sc as plsc

_N = 4096
_W = 4096
_NWORK = 32
_PER_W = _N // _NWORK
_C = 4
_NCHUNK = _PER_W // _C
_L = 16


def _layer_body(x_hbm, idx0_hbm, idx1_hbm, mmb_hbm, out_hbm,
                idx0_v, idx1_v, mmb_v, a_v, b_v, o_v,
                sem_a0, sem_a1, sem_b0, sem_b1, sem_o0, sem_o1):
    wid = lax.axis_index("s") * 2 + lax.axis_index("c")
    pltpu.sync_copy(idx0_hbm.at[wid], idx0_v)
    pltpu.sync_copy(idx1_hbm.at[wid], idx1_v)
    pltpu.sync_copy(mmb_hbm.at[wid], mmb_v)
    sem_a = (sem_a0, sem_a1)
    sem_b = (sem_b0, sem_b1)
    sem_o = (sem_o0, sem_o1)

    def gather(g, p):
        pltpu.async_copy(x_hbm.at[idx0_v.at[g]], a_v.at[p], sem_a[p])
        pltpu.async_copy(x_hbm.at[idx1_v.at[g]], b_v.at[p], sem_b[p])

    def compute(g, p):
        pltpu.make_async_copy(x_hbm.at[pl.ds(0, _C)], a_v.at[p], sem_a[p]).wait()
        pltpu.make_async_copy(x_hbm.at[pl.ds(0, _C)], b_v.at[p], sem_b[p]).wait()
        for c in range(_C):
            mm = mmb_v[g * _C + c, :]
            nm = jnp.bitwise_not(mm)

            def wbody(w, _, c=c, mm=mm, nm=nm):
                sl = pl.ds(w * _L, _L)
                a = a_v[p, c, sl]
                b = b_v[p, c, sl]
                o_v[p, c, sl] = ((a ^ mm) & (b ^ mm)) ^ nm
                return 0

            lax.fori_loop(0, _W // _L, wbody, 0, unroll=8)
        base = wid * _PER_W + g * _C
        pltpu.async_copy(o_v.at[p], out_hbm.at[pl.ds(base, _C)], sem_o[p])

    gather(0, 0)

    def chunk_body(h, carry):
        g0 = h * 2
        for q in range(2):
            g = g0 + q
            p = q
            nxt = g + 1
            if q == 0:
                gather(nxt, 1)
            else:
                @pl.when(nxt < _NCHUNK)
                def _():
                    gather(nxt, 0)

            @pl.when(g >= 2)
            def _():
                pltpu.make_async_copy(
                    o_v.at[p], out_hbm.at[pl.ds(0, _C)], sem_o[p]).wait()

            compute(g, p)
        return carry

    lax.fori_loop(0, _NCHUNK // 2, chunk_body, 0)
    pltpu.make_async_copy(o_v.at[0], out_hbm.at[pl.ds(0, _C)], sem_o[0]).wait()
    pltpu.make_async_copy(o_v.at[1], out_hbm.at[pl.ds(0, _C)], sem_o[1]).wait()


_sc_layer = functools.partial(
    pl.kernel,
    mesh=plsc.VectorSubcoreMesh(core_axis_name="c", subcore_axis_name="s"),
    out_type=jax.ShapeDtypeStruct((_N, _W), jnp.int32),
    scratch_types=[
        pltpu.VMEM((_NCHUNK, _C), jnp.int32),
        pltpu.VMEM((_NCHUNK, _C), jnp.int32),
        pltpu.VMEM((_PER_W, _L), jnp.int32),
        pltpu.VMEM((2, _C, _W), jnp.int32),
        pltpu.VMEM((2, _C, _W), jnp.int32),
        pltpu.VMEM((2, _C, _W), jnp.int32),
        pltpu.SemaphoreType.DMA,
        pltpu.SemaphoreType.DMA,
        pltpu.SemaphoreType.DMA,
        pltpu.SemaphoreType.DMA,
        pltpu.SemaphoreType.DMA,
        pltpu.SemaphoreType.DMA,
    ],
)(_layer_body)


def _gather_combine(x, idx0, idx1, nor_mask):
    idx0c = idx0.reshape(_NWORK, _NCHUNK, _C)
    idx1c = idx1.reshape(_NWORK, _NCHUNK, _C)
    mm = jnp.where(nor_mask, jnp.int32(-1), jnp.int32(0))
    mmb = jnp.broadcast_to(mm[:, None], (_N, _L)).reshape(_NWORK, _PER_W, _L)
    return _sc_layer(x, idx0c, idx1c, mmb)


def kernel(input_bitarrays, output_shape, adj_logits_0, nor_logits_0, adj_temp_0, nor_temp_0, adj_logits_1, nor_logits_1, adj_temp_1, nor_temp_1, adj_logits_2, nor_logits_2, adj_temp_2, nor_temp_2, adj_logits_3, nor_logits_3, adj_temp_3, nor_temp_3, adj_logits_4, nor_logits_4, adj_temp_4, nor_temp_4):
    params = {
        'adj_logits_0': adj_logits_0, 'nor_logits_0': nor_logits_0,
        'adj_temp_0': adj_temp_0, 'nor_temp_0': nor_temp_0,
        'adj_logits_1': adj_logits_1, 'nor_logits_1': nor_logits_1,
        'adj_temp_1': adj_temp_1, 'nor_temp_1': nor_temp_1,
        'adj_logits_2': adj_logits_2, 'nor_logits_2': nor_logits_2,
        'adj_temp_2': adj_temp_2, 'nor_temp_2': nor_temp_2,
        'adj_logits_3': adj_logits_3, 'nor_logits_3': nor_logits_3,
        'adj_temp_3': adj_temp_3, 'nor_temp_3': nor_temp_3,
        'adj_logits_4': adj_logits_4, 'nor_logits_4': nor_logits_4,
        'adj_temp_4': adj_temp_4, 'nor_temp_4': nor_temp_4,
    }
    key = jax.random.key(42)
    draws = []
    for i in range(5):
        al = params[f'adj_logits_{i}']
        at = params[f'adj_temp_{i}']
        nl = params[f'nor_logits_{i}']
        nt = params[f'nor_temp_{i}']
        k = jax.random.fold_in(key, i)
        k1, k2, k3 = jax.random.split(k, 3)
        logits = al * at
        idx0 = jax.random.categorical(k1, logits, axis=-1).astype(jnp.int32)
        idx1 = jax.random.categorical(k2, logits, axis=-1).astype(jnp.int32)
        nor_mask = jax.random.bernoulli(k3, jax.nn.sigmoid(nl * nt))
        draws.append((idx0, idx1, nor_mask))
    x = jnp.bitwise_or(input_bitarrays, jnp.int32(0) * output_shape)
    for idx0, idx1, nor_mask in draws:
        x = _gather_combine(x, idx0, idx1, nor_mask)
    return x


# speedup vs baseline: 1.1663x; 1.0098x over previous
import functools

import jax
import jax.numpy as jnp
from jax import lax
from jax.experimental import pallas as pl
from jax.experimental.pallas import tpu as pltpu
from jax.experimental.pallas import tpu_sc as plsc

_N = 4096
_W = 4096
_NWORK = 32
_PER_W = _N // _NWORK
_C = 4
_NCHUNK = _PER_W // _C
_L = 16


def _layer_body(x_hbm, idx0_hbm, idx1_hbm, mmb_hbm, out_hbm,
                idx0_v, idx1_v, mmb_v, a_v, b_v,
                sem_a0, sem_a1, sem_a2, sem_b0, sem_b1, sem_b2,
                sem_o0, sem_o1, sem_o2):
    wid = lax.axis_index("s") * 2 + lax.axis_index("c")
    pltpu.sync_copy(idx0_hbm.at[wid], idx0_v)
    pltpu.sync_copy(idx1_hbm.at[wid], idx1_v)
    pltpu.sync_copy(mmb_hbm.at[wid], mmb_v)
    sem_a = (sem_a0, sem_a1, sem_a2)
    sem_b = (sem_b0, sem_b1, sem_b2)
    sem_o = (sem_o0, sem_o1, sem_o2)

    def gather(g, p):
        pltpu.async_copy(x_hbm.at[idx0_v.at[g]], a_v.at[p], sem_a[p])
        pltpu.async_copy(x_hbm.at[idx1_v.at[g]], b_v.at[p], sem_b[p])

    def compute(g, p):
        pltpu.make_async_copy(x_hbm.at[pl.ds(0, _C)], a_v.at[p], sem_a[p]).wait()
        pltpu.make_async_copy(x_hbm.at[pl.ds(0, _C)], b_v.at[p], sem_b[p]).wait()
        for c in range(_C):
            mm = mmb_v[g * _C + c, :]
            nm = jnp.bitwise_not(mm)

            def wbody(w, _, c=c, mm=mm, nm=nm):
                sl = pl.ds(w * _L, _L)
                a = a_v[p, c, sl]
                b = b_v[p, c, sl]
                a_v[p, c, sl] = ((a ^ mm) & (b ^ mm)) ^ nm
                return 0

            lax.fori_loop(0, _W // _L, wbody, 0, unroll=8)
        base = wid * _PER_W + g * _C
        pltpu.async_copy(a_v.at[p], out_hbm.at[pl.ds(base, _C)], sem_o[p])

    gather(0, 0)
    gather(1, 1)

    def chunk_body(h, carry):
        for q in range(3):
            g = h * 3 + q
            nxt = g + 2
            pn = (q + 2) % 3

            @pl.when(nxt < _NCHUNK)
            def _():
                @pl.when(nxt >= 3)
                def _():
                    pltpu.make_async_copy(
                        a_v.at[pn], out_hbm.at[pl.ds(0, _C)], sem_o[pn]).wait()
                gather(nxt, pn)

            @pl.when(g < _NCHUNK)
            def _():
                compute(g, q)
        return carry

    lax.fori_loop(0, (_NCHUNK + 2) // 3, chunk_body, 0)
    for p in range(3):
        pltpu.make_async_copy(
            a_v.at[p], out_hbm.at[pl.ds(0, _C)], sem_o[p]).wait()


_sc_layer = functools.partial(
    pl.kernel,
    mesh=plsc.VectorSubcoreMesh(core_axis_name="c", subcore_axis_name="s"),
    out_type=jax.ShapeDtypeStruct((_N, _W), jnp.int32),
    scratch_types=[
        pltpu.VMEM((_NCHUNK, _C), jnp.int32),
        pltpu.VMEM((_NCHUNK, _C), jnp.int32),
        pltpu.VMEM((_PER_W, _L), jnp.int32),
        pltpu.VMEM((3, _C, _W), jnp.int32),
        pltpu.VMEM((3, _C, _W), jnp.int32),
        pltpu.SemaphoreType.DMA,
        pltpu.SemaphoreType.DMA,
        pltpu.SemaphoreType.DMA,
        pltpu.SemaphoreType.DMA,
        pltpu.SemaphoreType.DMA,
        pltpu.SemaphoreType.DMA,
        pltpu.SemaphoreType.DMA,
        pltpu.SemaphoreType.DMA,
        pltpu.SemaphoreType.DMA,
    ],
)(_layer_body)


def _gather_combine(x, idx0, idx1, nor_mask):
    idx0c = idx0.reshape(_NWORK, _NCHUNK, _C)
    idx1c = idx1.reshape(_NWORK, _NCHUNK, _C)
    mm = jnp.where(nor_mask, jnp.int32(-1), jnp.int32(0))
    mmb = jnp.broadcast_to(mm[:, None], (_N, _L)).reshape(_NWORK, _PER_W, _L)
    return _sc_layer(x, idx0c, idx1c, mmb)


def kernel(input_bitarrays, output_shape, adj_logits_0, nor_logits_0, adj_temp_0, nor_temp_0, adj_logits_1, nor_logits_1, adj_temp_1, nor_temp_1, adj_logits_2, nor_logits_2, adj_temp_2, nor_temp_2, adj_logits_3, nor_logits_3, adj_temp_3, nor_temp_3, adj_logits_4, nor_logits_4, adj_temp_4, nor_temp_4):
    params = {
        'adj_logits_0': adj_logits_0, 'nor_logits_0': nor_logits_0,
        'adj_temp_0': adj_temp_0, 'nor_temp_0': nor_temp_0,
        'adj_logits_1': adj_logits_1, 'nor_logits_1': nor_logits_1,
        'adj_temp_1': adj_temp_1, 'nor_temp_1': nor_temp_1,
        'adj_logits_2': adj_logits_2, 'nor_logits_2': nor_logits_2,
        'adj_temp_2': adj_temp_2, 'nor_temp_2': nor_temp_2,
        'adj_logits_3': adj_logits_3, 'nor_logits_3': nor_logits_3,
        'adj_temp_3': adj_temp_3, 'nor_temp_3': nor_temp_3,
        'adj_logits_4': adj_logits_4, 'nor_logits_4': nor_logits_4,
        'adj_temp_4': adj_temp_4, 'nor_temp_4': nor_temp_4,
    }
    key = jax.random.key(42)
    draws = []
    for i in range(5):
        al = params[f'adj_logits_{i}']
        at = params[f'adj_temp_{i}']
        nl = params[f'nor_logits_{i}']
        nt = params[f'nor_temp_{i}']
        k = jax.random.fold_in(key, i)
        k1, k2, k3 = jax.random.split(k, 3)
        logits = al * at
        idx0 = jax.random.categorical(k1, logits, axis=-1).astype(jnp.int32)
        idx1 = jax.random.categorical(k2, logits, axis=-1).astype(jnp.int32)
        nor_mask = jax.random.bernoulli(k3, jax.nn.sigmoid(nl * nt))
        draws.append((idx0, idx1, nor_mask))
    x = jnp.bitwise_or(input_bitarrays, jnp.int32(0) * output_shape)
    for idx0, idx1, nor_mask in draws:
        x = _gather_combine(x, idx0, idx1, nor_mask)
    return x
